# baseline (speedup 1.0000x reference)
import functools

import jax
import jax.numpy as jnp
from jax import lax
from jax.experimental import pallas as pl
from jax.experimental.pallas import tpu as pltpu
from jax.experimental.pallas import tpu_sc as plsc

VOCAB = 1000000
EMBED_DIM = 32
BATCH = 4096
SEQ = 200

NC = 2
NS = 16
NW = NC * NS

BT = BATCH // NW
DT = EMBED_DIM // 8

_mesh = plsc.VectorSubcoreMesh(
    core_axis_name="c", subcore_axis_name="s", num_cores=NC, num_subcores=NS
)


@functools.partial(
    pl.kernel,
    out_type=jax.ShapeDtypeStruct((SEQ, DT, NW, 8, BT), jnp.float32),
    mesh=_mesh,
    scratch_types=[
        pltpu.VMEM((SEQ, BT), jnp.int32),
        [pltpu.VMEM((BT, EMBED_DIM), jnp.float32) for _ in range(8)],
        [pltpu.VMEM((DT, 8, BT), jnp.float32) for _ in range(2)],
        [pltpu.SemaphoreType.DMA for _ in range(8)],
        [pltpu.SemaphoreType.DMA for _ in range(2)],
    ],
    compiler_params=pltpu.CompilerParams(
        use_tc_tiling_on_sc=False, needs_layout_passes=False
    ),
)
def _embed_sc(idx_hbm, table_hbm, out_hbm, idx_v, rows, tiles, semg, semo):
    wid = lax.axis_index("s") * NC + lax.axis_index("c")
    pltpu.sync_copy(idx_hbm.at[wid], idx_v)

    def fire_gather(s, rows, sem):
        pltpu.async_copy(table_hbm.at[idx_v.at[s]], rows, sem)

    def drain_gather(rows, sem):
        pltpu.make_async_copy(table_hbm.at[pl.ds(0, BT)], rows, sem).wait()

    def drain_out(tile, sem):
        pltpu.make_async_copy(out_hbm.at[0, :, 0], tile, sem).wait()

    lane = lax.iota(jnp.int32, 16)
    cvecs = [lane + jnp.full((16,), 16 * k, jnp.int32) for k in range(BT // 16)]

    def transpose(rows, tile):
        @plsc.parallel_loop(0, EMBED_DIM, unroll=8)
        def _(d):
            dvec = jnp.full((16,), 1, jnp.int32) * d
            dt = lax.shift_right_logical(d, 3)
            dl = lax.bitwise_and(d, 7)
            for k in range(BT // 16):
                tile[dt, dl, pl.ds(k * 16, 16)] = plsc.load_gather(
                    rows, [cvecs[k], dvec]
                )

    NB = 8
    for j in range(NB - 1):
        fire_gather(j, rows[j], semg[j])

    @pl.loop(0, SEQ, step=NB)
    def _ring(s):
        for b in range(NB):
            cur = s + b
            nb = (b + NB - 1) % NB

            @pl.when(cur + NB - 1 < SEQ)
            def _():
                fire_gather(cur + NB - 1, rows[nb], semg[nb])

            drain_gather(rows[b], semg[b])
            tl = tiles[b % 2]
            so = semo[b % 2]

            @pl.when(cur >= 2)
            def _():
                drain_out(tl, so)

            transpose(rows[b], tl)
            pltpu.async_copy(tl, out_hbm.at[cur, :, wid], so)

    drain_out(tiles[0], semo[0])
    drain_out(tiles[1], semo[1])


def kernel(x, table):
    idx = x.astype(jnp.int32).reshape(NW, BT, SEQ).transpose(0, 2, 1)
    out5 = _embed_sc(idx, table)
    return out5.transpose(2, 4, 0, 1, 3).reshape(BATCH, SEQ, EMBED_DIM)


# speedup vs baseline: 1.0194x; 1.0194x over previous
import functools

import jax
import jax.numpy as jnp
from jax import lax
from jax.experimental import pallas as pl
from jax.experimental.pallas import tpu as pltpu
from jax.experimental.pallas import tpu_sc as plsc

VOCAB = 1000000
EMBED_DIM = 32
BATCH = 4096
SEQ = 200

NC = 2
NS = 16
NW = NC * NS

BT = BATCH // NW
DT = EMBED_DIM // 8

_mesh = plsc.VectorSubcoreMesh(
    core_axis_name="c", subcore_axis_name="s", num_cores=NC, num_subcores=NS
)


@functools.partial(
    pl.kernel,
    out_type=jax.ShapeDtypeStruct((SEQ, DT, NW, 8, BT), jnp.float32),
    mesh=_mesh,
    scratch_types=[
        pltpu.VMEM((SEQ, BT), jnp.int32),
        [pltpu.VMEM((BT, EMBED_DIM), jnp.float32) for _ in range(8)],
        [pltpu.VMEM((DT, 8, BT), jnp.float32) for _ in range(4)],
        [pltpu.SemaphoreType.DMA for _ in range(8)],
        [pltpu.SemaphoreType.DMA for _ in range(4)],
    ],
    compiler_params=pltpu.CompilerParams(
        use_tc_tiling_on_sc=False, needs_layout_passes=False
    ),
)
def _embed_sc(idx_hbm, table_hbm, out_hbm, idx_v, rows, tiles, semg, semo):
    wid = lax.axis_index("s") * NC + lax.axis_index("c")
    pltpu.sync_copy(idx_hbm.at[wid], idx_v)

    def fire_gather(s, rows, sem):
        pltpu.async_copy(table_hbm.at[idx_v.at[s]], rows, sem)

    def drain_gather(rows, sem):
        pltpu.make_async_copy(table_hbm.at[pl.ds(0, BT)], rows, sem).wait()

    def drain_out(tile, sem):
        pltpu.make_async_copy(out_hbm.at[0, :, 0], tile, sem).wait()

    lane = lax.iota(jnp.int32, 16)
    cvecs = [lane + jnp.full((16,), 16 * k, jnp.int32) for k in range(BT // 16)]

    def transpose(rows, tile):
        @plsc.parallel_loop(0, EMBED_DIM, unroll=4)
        def _(d):
            dvec = jnp.full((16,), 1, jnp.int32) * d
            dt = lax.shift_right_logical(d, 3)
            dl = lax.bitwise_and(d, 7)
            for k in range(BT // 16):
                tile[dt, dl, pl.ds(k * 16, 16)] = plsc.load_gather(
                    rows, [cvecs[k], dvec]
                )

    NB = 8
    for j in range(NB - 1):
        fire_gather(j, rows[j], semg[j])

    @pl.loop(0, SEQ, step=NB)
    def _ring(s):
        for b in range(NB):
            cur = s + b
            nb = (b + NB - 1) % NB

            @pl.when(cur + NB - 1 < SEQ)
            def _():
                fire_gather(cur + NB - 1, rows[nb], semg[nb])

            drain_gather(rows[b], semg[b])
            tl = tiles[b % 4]
            so = semo[b % 4]

            @pl.when(cur >= 4)
            def _():
                drain_out(tl, so)

            transpose(rows[b], tl)
            pltpu.async_copy(tl, out_hbm.at[cur, :, wid], so)

    for q in range(4):
        drain_out(tiles[q], semo[q])


def kernel(x, table):
    idx = x.astype(jnp.int32).reshape(NW, BT, SEQ).transpose(0, 2, 1)
    out5 = _embed_sc(idx, table)
    return out5.transpose(2, 4, 0, 1, 3).reshape(BATCH, SEQ, EMBED_DIM)


# speedup vs baseline: 1.0325x; 1.0128x over previous
import functools

import jax
import jax.numpy as jnp
from jax import lax
from jax.experimental import pallas as pl
from jax.experimental.pallas import tpu as pltpu
from jax.experimental.pallas import tpu_sc as plsc

VOCAB = 1000000
EMBED_DIM = 32
BATCH = 4096
SEQ = 200

NC = 2
NS = 16
NW = NC * NS

BT = BATCH // NW
DT = EMBED_DIM // 8

_mesh = plsc.VectorSubcoreMesh(
    core_axis_name="c", subcore_axis_name="s", num_cores=NC, num_subcores=NS
)


@functools.partial(
    pl.kernel,
    out_type=jax.ShapeDtypeStruct((SEQ, DT, NW, 8, BT), jnp.float32),
    mesh=_mesh,
    scratch_types=[
        pltpu.VMEM((SEQ, BT), jnp.int32),
        [pltpu.VMEM((BT, 4 * EMBED_DIM), jnp.float32) for _ in range(4)],
        [pltpu.VMEM((DT, 8, BT), jnp.float32) for _ in range(2)],
        [pltpu.SemaphoreType.DMA for _ in range(8)],
        [pltpu.SemaphoreType.DMA for _ in range(2)],
    ],
    compiler_params=pltpu.CompilerParams(
        use_tc_tiling_on_sc=False, needs_layout_passes=False
    ),
)
def _embed_sc(idx_hbm, table_hbm, out_hbm, idx_v, rows, tiles, semg, semo):
    wid = lax.axis_index("s") * NC + lax.axis_index("c")
    pltpu.sync_copy(idx_hbm.at[wid], idx_v)

    def fire_gather(s, rows, sem):
        pltpu.async_copy(table_hbm.at[idx_v.at[s]], rows, sem)

    def drain_gather(rows, sem):
        pltpu.make_async_copy(table_hbm.at[pl.ds(0, BT)], rows, sem).wait()

    def drain_out(tile, sem):
        pltpu.make_async_copy(out_hbm.at[0, :, 0], tile, sem).wait()

    lane = lax.iota(jnp.int32, 16)
    cvecs = [lane + jnp.full((16,), 16 * k, jnp.int32) for k in range(BT // 16)]

    def transpose(rows, tile):
        @plsc.parallel_loop(0, EMBED_DIM, unroll=4)
        def _(d):
            dvec = jnp.full((16,), 1, jnp.int32) * d
            dt = lax.shift_right_logical(d, 3)
            dl = lax.bitwise_and(d, 7)
            for k in range(BT // 16):
                tile[dt, dl, pl.ds(k * 16, 16)] = plsc.load_gather(
                    rows, [cvecs[k], dvec]
                )

    NB = 4
    for j in range(NB - 1):
        fire_gather(j, rows[j], semg[j])

    @pl.loop(0, SEQ, step=NB)
    def _ring(s):
        for b in range(NB):
            cur = s + b
            nb = (b + NB - 1) % NB

            @pl.when(cur + NB - 1 < SEQ)
            def _():
                fire_gather(cur + NB - 1, rows[nb], semg[nb])

            drain_gather(rows[b], semg[b])
            tl = tiles[b % 2]
            so = semo[b % 2]

            @pl.when(cur >= 2)
            def _():
                drain_out(tl, so)

            transpose(rows[b], tl)
            pltpu.async_copy(tl, out_hbm.at[cur, :, wid], so)

    drain_out(tiles[0], semo[0])
    drain_out(tiles[1], semo[1])


def kernel(x, table):
    idx = x.astype(jnp.int32).reshape(NW, BT, SEQ).transpose(0, 2, 1)
    t128 = jnp.pad(table, ((0, 0), (0, 96)))
    out5 = _embed_sc(idx, t128)
    return out5.transpose(2, 4, 0, 1, 3).reshape(BATCH, SEQ, EMBED_DIM)


# speedup vs baseline: 1.0333x; 1.0008x over previous
import functools

import jax
import jax.numpy as jnp
from jax import lax
from jax.experimental import pallas as pl
from jax.experimental.pallas import tpu as pltpu
from jax.experimental.pallas import tpu_sc as plsc

VOCAB = 1000000
EMBED_DIM = 32
BATCH = 4096
SEQ = 200

NC = 2
NS = 16
NW = NC * NS

BT = BATCH // NW
DT = EMBED_DIM // 8

_mesh = plsc.VectorSubcoreMesh(
    core_axis_name="c", subcore_axis_name="s", num_cores=NC, num_subcores=NS
)


@functools.partial(
    pl.kernel,
    out_type=jax.ShapeDtypeStruct((SEQ, DT, NW, 8, BT), jnp.float32),
    mesh=_mesh,
    scratch_types=[
        pltpu.VMEM((SEQ, BT), jnp.int32),
        [pltpu.VMEM((BT, 4 * EMBED_DIM), jnp.float32) for _ in range(4)],
        [pltpu.VMEM((DT, 8, BT), jnp.float32) for _ in range(2)],
        [pltpu.SemaphoreType.DMA for _ in range(4)],
        [pltpu.SemaphoreType.DMA for _ in range(2)],
    ],
    compiler_params=pltpu.CompilerParams(
        use_tc_tiling_on_sc=False, needs_layout_passes=False
    ),
)
def _embed_sc(idx_hbm, table_hbm, out_hbm, idx_v, rows, tiles, semg, semo):
    wid = lax.axis_index("s") * NC + lax.axis_index("c")
    pltpu.sync_copy(idx_hbm.at[wid], idx_v)

    def fire_gather(s, rows, sem):
        pltpu.async_copy(table_hbm.at[idx_v.at[s]], rows, sem)

    def drain_gather(rows, sem):
        pltpu.make_async_copy(table_hbm.at[pl.ds(0, BT)], rows, sem).wait()

    def drain_out(tile, sem):
        pltpu.make_async_copy(out_hbm.at[0, :, 0], tile, sem).wait()

    lane = lax.iota(jnp.int32, 16)
    cvecs = [lane + jnp.full((16,), 16 * k, jnp.int32) for k in range(BT // 16)]

    def transpose(rows, tile):
        @plsc.parallel_loop(0, EMBED_DIM, unroll=4)
        def _(d):
            dvec = jnp.full((16,), 1, jnp.int32) * d
            dt = lax.shift_right_logical(d, 3)
            dl = lax.bitwise_and(d, 7)
            for k in range(BT // 16):
                tile[dt, dl, pl.ds(k * 16, 16)] = plsc.load_gather(
                    rows, [cvecs[k], dvec]
                )

    NB = 4
    for j in range(NB - 1):
        fire_gather(j, rows[j], semg[j])

    @pl.loop(0, SEQ, step=NB)
    def _ring(s):
        for b in range(NB):
            cur = s + b
            nb = (b + NB - 1) % NB

            @pl.when(cur + NB - 1 < SEQ)
            def _():
                fire_gather(cur + NB - 1, rows[nb], semg[nb])

            drain_gather(rows[b], semg[b])
            tl = tiles[b % 2]
            so = semo[b % 2]

            @pl.when(cur >= 2)
            def _():
                drain_out(tl, so)

            transpose(rows[b], tl)
            pltpu.async_copy(tl, out_hbm.at[cur, :, wid], so)

    drain_out(tiles[0], semo[0])
    drain_out(tiles[1], semo[1])


def kernel(x, table):
    idx = x.astype(jnp.int32).reshape(NW, BT, SEQ).transpose(0, 2, 1)
    t128 = jnp.pad(table, ((0, 0), (0, 96)))
    out5 = _embed_sc(idx, t128)
    return out5.transpose(2, 4, 0, 1, 3).reshape(BATCH, SEQ, EMBED_DIM)
